# baseline (speedup 1.0000x reference)
import functools

import jax
import jax.numpy as jnp
from jax import lax
from jax.experimental import pallas as pl
from jax.experimental.pallas import tpu as pltpu
from jax.experimental.pallas import tpu_sc as plsc

B = 512
D = 128
N = 100000
KP1 = 1024
EPS = 1e-7
MARGIN = 1.0
NBLK = 2000

NC = 2
NS = 16
NW = NC * NS
BPW = B // NW
ROWS = BPW * KP1 // 128

HIMASK = 0xFFFF0000


def _dist2(w, q):
    qn = q / (jnp.sqrt(jnp.sum(q * q, axis=1, keepdims=True)) + EPS)
    qnn = jnp.sum(qn * qn, axis=1)[None, :]
    s2 = lax.dot_general(w, qn + qn, (((1,), (1,)), ((), ())),
                         preferred_element_type=jnp.float32)
    wn = jnp.sum(w * w, axis=1, keepdims=True)
    return (wn + qnn) - s2


def _pack_body(l_ref, ab_ref, meml_ref, memab_ref, out_ref):
    za = _dist2(memab_ref[...], l_ref[...])
    zl = _dist2(meml_ref[...], ab_ref[...])
    ua = lax.bitcast_convert_type(za, jnp.uint32) >> 16
    ul = lax.bitcast_convert_type(zl, jnp.uint32) & jnp.uint32(HIMASK)
    w = ul | ua
    for q in range(B // 128):
        out_ref[q, :, :] = w[:, q * 128:(q + 1) * 128]


def _pack(l, ab, memory_l, memory_ab):
    out = pl.pallas_call(
        _pack_body,
        grid=(N // NBLK,),
        in_specs=[
            pl.BlockSpec((B, D), lambda i: (0, 0)),
            pl.BlockSpec((B, D), lambda i: (0, 0)),
            pl.BlockSpec((NBLK, D), lambda i: (i, 0)),
            pl.BlockSpec((NBLK, D), lambda i: (i, 0)),
        ],
        out_specs=pl.BlockSpec((B // 128, NBLK, 128), lambda i: (0, i, 0)),
        out_shape=jax.ShapeDtypeStruct((B // 128, N, 128), jnp.uint32),
    )(l, ab, memory_l, memory_ab)
    return out.reshape(N * B)


def _score_body(w_ref, idx_ref, ol_ref, oab_ref,
                idx_v, flat_v, g_v, ol_v, oab_v, sem0, sem1):
    wid = lax.axis_index("s") * NC + lax.axis_index("c")
    pltpu.sync_copy(idx_ref.at[pl.ds(wid * BPW, BPW)], idx_v)
    sems = (sem0, sem1)
    half = ROWS // 2

    def make_fire(sem):
        def fire(j, carry):
            b = wid * BPW + (j >> 3)
            cb = ((b >> 7) * N) * 128 + (b & 127)
            for t in range(8):
                v = idx_v[j >> 3, pl.ds(pl.multiple_of(((j & 7) << 7) + t * 16, 16), 16)]
                flat_v[j, pl.ds(t * 16, 16)] = (v << 7) + cb
            pltpu.async_copy(w_ref.at[flat_v.at[j]],
                             g_v.at[pl.ds(j * 128, 128)], sem)
            return carry
        return fire

    def sqrt16(x):
        x = jnp.maximum(x, 1e-20)
        i = jnp.int32(0x5F3759DF) - (lax.bitcast_convert_type(x, jnp.int32) >> 1)
        r = lax.bitcast_convert_type(i, jnp.float32)
        r = r * (1.5 - 0.5 * x * r * r)
        return x * r

    def compute(j, carry):
        for t in range(8):
            v = g_v[pl.ds(j * 128 + t * 16, 16)]
            da = sqrt16(lax.bitcast_convert_type(v << 16, jnp.float32))
            dl = sqrt16(lax.bitcast_convert_type(v & jnp.uint32(HIMASK),
                                                 jnp.float32))
            ra = jnp.maximum(MARGIN - da, 0.0)
            rl = jnp.maximum(MARGIN - dl, 0.0)
            if t == 0:
                posv = lax.iota(jnp.int32, 16) + ((j & 7) << 7)
                ra = jnp.where(posv == 0, da, ra)
                rl = jnp.where(posv == 0, dl, rl)
            ol_v[j >> 3, pl.ds(pl.multiple_of(((j & 7) << 7) + t * 16, 16), 16)] = ra
            oab_v[j >> 3, pl.ds(pl.multiple_of(((j & 7) << 7) + t * 16, 16), 16)] = rl
        return carry

    for h in range(2):
        lax.fori_loop(h * half, (h + 1) * half, make_fire(sems[h]), 0)
    for h in range(2):
        pltpu.make_async_copy(w_ref.at[pl.ds(0, half * 128)],
                              g_v.at[pl.ds(h * half * 128, half * 128)],
                              sems[h]).wait()
        lax.fori_loop(h * half, (h + 1) * half, compute, 0)
    pltpu.sync_copy(ol_v, ol_ref.at[pl.ds(wid * BPW, BPW)])
    pltpu.sync_copy(oab_v, oab_ref.at[pl.ds(wid * BPW, BPW)])


def _score(w_flat, idx4):
    mesh = plsc.VectorSubcoreMesh(core_axis_name="c", subcore_axis_name="s")
    f = pl.kernel(
        _score_body,
        out_type=(jax.ShapeDtypeStruct((B, KP1), jnp.float32),
                  jax.ShapeDtypeStruct((B, KP1), jnp.float32)),
        mesh=mesh,
        scratch_types=[
            pltpu.VMEM((BPW, KP1), jnp.int32),
            pltpu.VMEM((ROWS, 128), jnp.int32),
            pltpu.VMEM((ROWS * 128,), jnp.uint32),
            pltpu.VMEM((BPW, KP1), jnp.float32),
            pltpu.VMEM((BPW, KP1), jnp.float32),
            pltpu.SemaphoreType.DMA,
            pltpu.SemaphoreType.DMA,
        ],
    )
    return f(w_flat, idx4)


def kernel(l, ab, y, idx, memory_l, memory_ab):
    idx4 = idx.astype(jnp.int32)
    w_flat = _pack(l, ab, memory_l, memory_ab)
    ol, oab = _score(w_flat, idx4)
    return (ol.reshape(B, KP1, 1), oab.reshape(B, KP1, 1))


# speedup vs baseline: 1.1301x; 1.1301x over previous
import functools

import jax
import jax.numpy as jnp
from jax import lax
from jax.experimental import pallas as pl
from jax.experimental.pallas import tpu as pltpu
from jax.experimental.pallas import tpu_sc as plsc

B = 512
D = 128
N = 100000
KP1 = 1024
EPS = 1e-7
MARGIN = 1.0
NBLK = 4000

NC = 2
NS = 16
NW = NC * NS
BPW = B // NW
ROWS = BPW * KP1 // 128

HIMASK = 0xFFFF0000


def _dist2(w, q):
    qn = q / (jnp.sqrt(jnp.sum(q * q, axis=1, keepdims=True)) + EPS)
    qnn = jnp.sum(qn * qn, axis=1)[None, :]
    s2 = lax.dot_general(w, qn + qn, (((1,), (1,)), ((), ())),
                         preferred_element_type=jnp.float32)
    wn = jnp.sum(w * w, axis=1, keepdims=True)
    return (wn + qnn) - s2


def _pack_body(l_ref, ab_ref, meml_ref, memab_ref, out_ref):
    za = _dist2(memab_ref[...], l_ref[...])
    zl = _dist2(meml_ref[...], ab_ref[...])
    ua = lax.bitcast_convert_type(za, jnp.uint32) >> 16
    ul = lax.bitcast_convert_type(zl, jnp.uint32) & jnp.uint32(HIMASK)
    w = ul | ua
    for q in range(B // 128):
        out_ref[q, :, :] = w[:, q * 128:(q + 1) * 128]


def _pack(l, ab, memory_l, memory_ab):
    out = pl.pallas_call(
        _pack_body,
        grid=(N // NBLK,),
        in_specs=[
            pl.BlockSpec((B, D), lambda i: (0, 0)),
            pl.BlockSpec((B, D), lambda i: (0, 0)),
            pl.BlockSpec((NBLK, D), lambda i: (i, 0)),
            pl.BlockSpec((NBLK, D), lambda i: (i, 0)),
        ],
        out_specs=pl.BlockSpec((B // 128, NBLK, 128), lambda i: (0, i, 0)),
        out_shape=jax.ShapeDtypeStruct((B // 128, N, 128), jnp.uint32),
    )(l, ab, memory_l, memory_ab)
    return out.reshape(N * B)


def _score_body(w_ref, idx_ref, ol_ref, oab_ref,
                idx_v, flat_v, g_v, ol_v, oab_v, sem0, sem1):
    wid = lax.axis_index("s") * NC + lax.axis_index("c")
    pltpu.sync_copy(idx_ref.at[wid], idx_v)
    sems = (sem0, sem1)
    half = ROWS // 2

    def make_fire(sem):
        def fire(j, carry):
            b = wid * BPW + (j >> 3)
            cb = ((b >> 7) * N) * 128 + (b & 127)
            for t in range(8):
                v = idx_v[j, pl.ds(t * 16, 16)]
                flat_v[j, pl.ds(t * 16, 16)] = (v << 7) + cb
            pltpu.async_copy(w_ref.at[flat_v.at[j]],
                             g_v.at[pl.ds(j * 128, 128)], sem)
            return carry
        return fire

    def sqrt16(x):
        x = jnp.maximum(x, 1e-20)
        i = jnp.int32(0x5F3759DF) - (lax.bitcast_convert_type(x, jnp.int32) >> 1)
        r = lax.bitcast_convert_type(i, jnp.float32)
        r = r * (1.5 - 0.5 * x * r * r)
        return x * r

    def compute(j, carry):
        for t in range(8):
            v = g_v[pl.ds(j * 128 + t * 16, 16)]
            da = sqrt16(lax.bitcast_convert_type(v << 16, jnp.float32))
            dl = sqrt16(lax.bitcast_convert_type(v & jnp.uint32(HIMASK),
                                                 jnp.float32))
            ra = jnp.maximum(MARGIN - da, 0.0)
            rl = jnp.maximum(MARGIN - dl, 0.0)
            if t == 0:
                posv = lax.iota(jnp.int32, 16) + ((j & 7) << 7)
                ra = jnp.where(posv == 0, da, ra)
                rl = jnp.where(posv == 0, dl, rl)
            ol_v[j, pl.ds(t * 16, 16)] = ra
            oab_v[j, pl.ds(t * 16, 16)] = rl
        return carry

    for h in range(2):
        lax.fori_loop(h * half, (h + 1) * half, make_fire(sems[h]), 0)
    for h in range(2):
        pltpu.make_async_copy(w_ref.at[pl.ds(0, half * 128)],
                              g_v.at[pl.ds(h * half * 128, half * 128)],
                              sems[h]).wait()
        lax.fori_loop(h * half, (h + 1) * half, compute, 0)
    pltpu.sync_copy(ol_v, ol_ref.at[wid])
    pltpu.sync_copy(oab_v, oab_ref.at[wid])


def _score(w_flat, idx4):
    mesh = plsc.VectorSubcoreMesh(core_axis_name="c", subcore_axis_name="s")
    f = pl.kernel(
        _score_body,
        out_type=(jax.ShapeDtypeStruct((NW, ROWS, 128), jnp.float32),
                  jax.ShapeDtypeStruct((NW, ROWS, 128), jnp.float32)),
        mesh=mesh,
        scratch_types=[
            pltpu.VMEM((ROWS, 128), jnp.int32),
            pltpu.VMEM((ROWS, 128), jnp.int32),
            pltpu.VMEM((ROWS * 128,), jnp.uint32),
            pltpu.VMEM((ROWS, 128), jnp.float32),
            pltpu.VMEM((ROWS, 128), jnp.float32),
            pltpu.SemaphoreType.DMA,
            pltpu.SemaphoreType.DMA,
        ],
    )
    return f(w_flat, idx4)


def kernel(l, ab, y, idx, memory_l, memory_ab):
    idx4 = idx.astype(jnp.int32).reshape(NW, ROWS, 128)
    w_flat = _pack(l, ab, memory_l, memory_ab)
    ol, oab = _score(w_flat, idx4)
    return (ol.reshape(B, KP1, 1), oab.reshape(B, KP1, 1))


# speedup vs baseline: 1.1511x; 1.0186x over previous
import functools

import jax
import jax.numpy as jnp
from jax import lax
from jax.experimental import pallas as pl
from jax.experimental.pallas import tpu as pltpu
from jax.experimental.pallas import tpu_sc as plsc

B = 512
D = 128
N = 100000
KP1 = 1024
EPS = 1e-7
MARGIN = 1.0
NBLK = 5000

NC = 2
NS = 16
NW = NC * NS
BPW = B // NW
ROWS = BPW * KP1 // 128

HIMASK = 0xFFFF0000


def _dist2(w, q):
    qn = q / (jnp.sqrt(jnp.sum(q * q, axis=1, keepdims=True)) + EPS)
    qnn = jnp.sum(qn * qn, axis=1)[None, :]
    s2 = lax.dot_general(w, qn + qn, (((1,), (1,)), ((), ())),
                         preferred_element_type=jnp.float32)
    wn = jnp.sum(w * w, axis=1, keepdims=True)
    return (wn + qnn) - s2


def _pack_body(l_ref, ab_ref, meml_ref, memab_ref, out_ref):
    za = _dist2(memab_ref[...], l_ref[...])
    zl = _dist2(meml_ref[...], ab_ref[...])
    ua = lax.bitcast_convert_type(za, jnp.uint32) >> 16
    ul = lax.bitcast_convert_type(zl, jnp.uint32) & jnp.uint32(HIMASK)
    w = ul | ua
    for q in range(B // 128):
        out_ref[q, :, :] = w[:, q * 128:(q + 1) * 128]


def _pack(l, ab, memory_l, memory_ab):
    out = pl.pallas_call(
        _pack_body,
        grid=(N // NBLK,),
        in_specs=[
            pl.BlockSpec((B, D), lambda i: (0, 0)),
            pl.BlockSpec((B, D), lambda i: (0, 0)),
            pl.BlockSpec((NBLK, D), lambda i: (i, 0)),
            pl.BlockSpec((NBLK, D), lambda i: (i, 0)),
        ],
        out_specs=pl.BlockSpec((B // 128, NBLK, 128), lambda i: (0, i, 0)),
        out_shape=jax.ShapeDtypeStruct((B // 128, N, 128), jnp.uint32),
    )(l, ab, memory_l, memory_ab)
    return out.reshape(N * B)


def _score_body(w_ref, idx_ref, ol_ref, oab_ref,
                idx_v, flat_v, g_v, ol_v, oab_v, sem0, sem1):
    wid = lax.axis_index("s") * NC + lax.axis_index("c")
    pltpu.sync_copy(idx_ref.at[wid], idx_v)
    sems = (sem0, sem1)
    half = ROWS // 2

    def make_fire(sem):
        def fire(j, carry):
            b = wid * BPW + (j >> 3)
            cb = ((b >> 7) * N) * 128 + (b & 127)
            for t in range(8):
                v = idx_v[j, pl.ds(t * 16, 16)]
                flat_v[j, pl.ds(t * 16, 16)] = (v << 7) + cb
            pltpu.async_copy(w_ref.at[flat_v.at[j]],
                             g_v.at[pl.ds(j * 128, 128)], sem)
            return carry
        return fire

    def sqrt16(x):
        x = jnp.maximum(x, 1e-20)
        i = jnp.int32(0x5F3759DF) - (lax.bitcast_convert_type(x, jnp.int32) >> 1)
        r = lax.bitcast_convert_type(i, jnp.float32)
        r = r * (1.5 - 0.5 * x * r * r)
        return x * r

    def compute(j, carry):
        for t in range(8):
            v = g_v[pl.ds(j * 128 + t * 16, 16)]
            da = sqrt16(lax.bitcast_convert_type(v << 16, jnp.float32))
            dl = sqrt16(lax.bitcast_convert_type(v & jnp.uint32(HIMASK),
                                                 jnp.float32))
            ra = jnp.maximum(MARGIN - da, 0.0)
            rl = jnp.maximum(MARGIN - dl, 0.0)
            if t == 0:
                posv = lax.iota(jnp.int32, 16) + ((j & 7) << 7)
                ra = jnp.where(posv == 0, da, ra)
                rl = jnp.where(posv == 0, dl, rl)
            ol_v[j, pl.ds(t * 16, 16)] = ra
            oab_v[j, pl.ds(t * 16, 16)] = rl
        return carry

    for h in range(2):
        lax.fori_loop(h * half, (h + 1) * half, make_fire(sems[h]), 0)
    for h in range(2):
        pltpu.make_async_copy(w_ref.at[pl.ds(0, half * 128)],
                              g_v.at[pl.ds(h * half * 128, half * 128)],
                              sems[h]).wait()
        lax.fori_loop(h * half, (h + 1) * half, compute, 0)
    pltpu.sync_copy(ol_v, ol_ref.at[wid])
    pltpu.sync_copy(oab_v, oab_ref.at[wid])


def _score(w_flat, idx4):
    mesh = plsc.VectorSubcoreMesh(core_axis_name="c", subcore_axis_name="s")
    f = pl.kernel(
        _score_body,
        out_type=(jax.ShapeDtypeStruct((NW, ROWS, 128), jnp.float32),
                  jax.ShapeDtypeStruct((NW, ROWS, 128), jnp.float32)),
        mesh=mesh,
        scratch_types=[
            pltpu.VMEM((ROWS, 128), jnp.int32),
            pltpu.VMEM((ROWS, 128), jnp.int32),
            pltpu.VMEM((ROWS * 128,), jnp.uint32),
            pltpu.VMEM((ROWS, 128), jnp.float32),
            pltpu.VMEM((ROWS, 128), jnp.float32),
            pltpu.SemaphoreType.DMA,
            pltpu.SemaphoreType.DMA,
        ],
    )
    return f(w_flat, idx4)


def kernel(l, ab, y, idx, memory_l, memory_ab):
    idx4 = idx.astype(jnp.int32).reshape(NW, ROWS, 128)
    w_flat = _pack(l, ab, memory_l, memory_ab)
    ol, oab = _score(w_flat, idx4)
    return (ol.reshape(B, KP1, 1), oab.reshape(B, KP1, 1))


# speedup vs baseline: 1.1567x; 1.0049x over previous
import functools

import jax
import jax.numpy as jnp
from jax import lax
from jax.experimental import pallas as pl
from jax.experimental.pallas import tpu as pltpu
from jax.experimental.pallas import tpu_sc as plsc

B = 512
D = 128
N = 100000
KP1 = 1024
EPS = 1e-7
MARGIN = 1.0
NBLK = 5000

NC = 2
NS = 16
NW = NC * NS
BPW = B // NW
ROWS = BPW * KP1 // 128

HIMASK = 0xFFFF0000


def _dist2(w, q):
    qn = q / (jnp.sqrt(jnp.sum(q * q, axis=1, keepdims=True)) + EPS)
    qnn = jnp.sum(qn * qn, axis=1)[None, :]
    s2 = lax.dot_general(w, qn + qn, (((1,), (1,)), ((), ())),
                         preferred_element_type=jnp.float32)
    wn = jnp.sum(w * w, axis=1, keepdims=True)
    return (wn + qnn) - s2


def _pack_body(l_ref, ab_ref, meml_ref, memab_ref, out_ref):
    za = _dist2(memab_ref[...], l_ref[...])
    zl = _dist2(meml_ref[...], ab_ref[...])
    ua = lax.bitcast_convert_type(za, jnp.uint32) >> 16
    ul = lax.bitcast_convert_type(zl, jnp.uint32) & jnp.uint32(HIMASK)
    w = ul | ua
    for q in range(B // 128):
        out_ref[q, :, :] = w[:, q * 128:(q + 1) * 128]


def _pack(l, ab, memory_l, memory_ab):
    out = pl.pallas_call(
        _pack_body,
        grid=(N // NBLK,),
        in_specs=[
            pl.BlockSpec((B, D), lambda i: (0, 0)),
            pl.BlockSpec((B, D), lambda i: (0, 0)),
            pl.BlockSpec((NBLK, D), lambda i: (i, 0)),
            pl.BlockSpec((NBLK, D), lambda i: (i, 0)),
        ],
        out_specs=pl.BlockSpec((B // 128, NBLK, 128), lambda i: (0, i, 0)),
        out_shape=jax.ShapeDtypeStruct((B // 128, N, 128), jnp.uint32),
    )(l, ab, memory_l, memory_ab)
    return out.reshape(N * B)


def _score_body(w_ref, idx_ref, ol_ref, oab_ref,
                idx_v, flat_v, g_v, ol_v, oab_v, sem0, sem1, sem2, sem3):
    wid = lax.axis_index("s") * NC + lax.axis_index("c")
    pltpu.sync_copy(idx_ref.at[wid], idx_v)
    sems = (sem0, sem1, sem2, sem3)
    half = ROWS // 4

    def make_fire(sem):
        def fire(j, carry):
            b = wid * BPW + (j >> 3)
            cb = ((b >> 7) * N) * 128 + (b & 127)
            for t in range(8):
                v = idx_v[j, pl.ds(t * 16, 16)]
                flat_v[j, pl.ds(t * 16, 16)] = (v << 7) + cb
            pltpu.async_copy(w_ref.at[flat_v.at[j]],
                             g_v.at[pl.ds(j * 128, 128)], sem)
            return carry
        return fire

    def sqrt16(x):
        x = jnp.maximum(x, 1e-20)
        i = jnp.int32(0x5F3759DF) - (lax.bitcast_convert_type(x, jnp.int32) >> 1)
        r = lax.bitcast_convert_type(i, jnp.float32)
        r = r * (1.5 - 0.5 * x * r * r)
        return x * r

    def compute(j, carry):
        for t in range(8):
            v = g_v[pl.ds(j * 128 + t * 16, 16)]
            da = sqrt16(lax.bitcast_convert_type(v << 16, jnp.float32))
            dl = sqrt16(lax.bitcast_convert_type(v & jnp.uint32(HIMASK),
                                                 jnp.float32))
            ra = jnp.maximum(MARGIN - da, 0.0)
            rl = jnp.maximum(MARGIN - dl, 0.0)
            if t == 0:
                posv = lax.iota(jnp.int32, 16) + ((j & 7) << 7)
                ra = jnp.where(posv == 0, da, ra)
                rl = jnp.where(posv == 0, dl, rl)
            ol_v[j, pl.ds(t * 16, 16)] = ra
            oab_v[j, pl.ds(t * 16, 16)] = rl
        return carry

    for h in range(4):
        lax.fori_loop(h * half, (h + 1) * half, make_fire(sems[h]), 0)
    for h in range(4):
        pltpu.make_async_copy(w_ref.at[pl.ds(0, half * 128)],
                              g_v.at[pl.ds(h * half * 128, half * 128)],
                              sems[h]).wait()
        lax.fori_loop(h * half, (h + 1) * half, compute, 0)
    pltpu.sync_copy(ol_v, ol_ref.at[wid])
    pltpu.sync_copy(oab_v, oab_ref.at[wid])


def _score(w_flat, idx4):
    mesh = plsc.VectorSubcoreMesh(core_axis_name="c", subcore_axis_name="s")
    f = pl.kernel(
        _score_body,
        out_type=(jax.ShapeDtypeStruct((NW, ROWS, 128), jnp.float32),
                  jax.ShapeDtypeStruct((NW, ROWS, 128), jnp.float32)),
        mesh=mesh,
        scratch_types=[
            pltpu.VMEM((ROWS, 128), jnp.int32),
            pltpu.VMEM((ROWS, 128), jnp.int32),
            pltpu.VMEM((ROWS * 128,), jnp.uint32),
            pltpu.VMEM((ROWS, 128), jnp.float32),
            pltpu.VMEM((ROWS, 128), jnp.float32),
            pltpu.SemaphoreType.DMA,
            pltpu.SemaphoreType.DMA,
            pltpu.SemaphoreType.DMA,
            pltpu.SemaphoreType.DMA,
        ],
    )
    return f(w_flat, idx4)


def kernel(l, ab, y, idx, memory_l, memory_ab):
    idx4 = idx.astype(jnp.int32).reshape(NW, ROWS, 128)
    w_flat = _pack(l, ab, memory_l, memory_ab)
    ol, oab = _score(w_flat, idx4)
    return (ol.reshape(B, KP1, 1), oab.reshape(B, KP1, 1))


# speedup vs baseline: 1.1575x; 1.0007x over previous
import jax
import jax.numpy as jnp
from jax import lax
from jax.experimental import pallas as pl
from jax.experimental.pallas import tpu as pltpu
from jax.experimental.pallas import tpu_sc as plsc

B = 512
D = 128
N = 100000
KP1 = 1024
EPS = 1e-7
MARGIN = 1.0
NBLK = 5000

NC = 2
NS = 16
NW = NC * NS
BPW = B // NW
ROWS = BPW * KP1 // 128

HIMASK = 0xFFFF0000


def _dist2(w, q):
    qn = q / (jnp.sqrt(jnp.sum(q * q, axis=1, keepdims=True)) + EPS)
    qnn = jnp.sum(qn * qn, axis=1)[None, :]
    s2 = lax.dot_general(w, qn + qn, (((1,), (1,)), ((), ())),
                         preferred_element_type=jnp.float32)
    wn = jnp.sum(w * w, axis=1, keepdims=True)
    return (wn + qnn) - s2


def _pack_body(l_ref, ab_ref, meml_ref, memab_ref, out_ref):
    za = _dist2(memab_ref[...], l_ref[...])
    zl = _dist2(meml_ref[...], ab_ref[...])
    ua = lax.bitcast_convert_type(za, jnp.uint32) >> 16
    ul = lax.bitcast_convert_type(zl, jnp.uint32) & jnp.uint32(HIMASK)
    w = ul | ua
    for q in range(B // 128):
        out_ref[q, :, :] = w[:, q * 128:(q + 1) * 128]


def _pack(l, ab, memory_l, memory_ab):
    out = pl.pallas_call(
        _pack_body,
        grid=(N // NBLK,),
        in_specs=[
            pl.BlockSpec((B, D), lambda i: (0, 0)),
            pl.BlockSpec((B, D), lambda i: (0, 0)),
            pl.BlockSpec((NBLK, D), lambda i: (i, 0)),
            pl.BlockSpec((NBLK, D), lambda i: (i, 0)),
        ],
        out_specs=pl.BlockSpec((B // 128, NBLK, 128), lambda i: (0, i, 0)),
        out_shape=jax.ShapeDtypeStruct((B // 128, N, 128), jnp.uint32),
    )(l, ab, memory_l, memory_ab)
    return out.reshape(N * B)


def _score_body(w_ref, idx_ref, ol_ref, oab_ref,
                idx_v, flat_v, g_v, ol_v, oab_v, sem0, sem1, sem2, sem3):
    wid = lax.axis_index("s") * NC + lax.axis_index("c")
    pltpu.sync_copy(idx_ref.at[wid], idx_v)
    sems = (sem0, sem1, sem2, sem3)
    half = ROWS // 4

    def make_fire(sem):
        def fire(j, carry):
            b = wid * BPW + (j >> 3)
            cb = ((b >> 7) * N) * 128 + (b & 127)
            for t in range(8):
                v = idx_v[j, pl.ds(t * 16, 16)]
                flat_v[j, pl.ds(t * 16, 16)] = (v << 7) + cb
            pltpu.async_copy(w_ref.at[flat_v.at[j]],
                             g_v.at[pl.ds(j * 128, 128)], sem)
            return carry
        return fire

    def sqrt16(x):
        x = jnp.maximum(x, 1e-20)
        i = jnp.int32(0x5F3759DF) - (lax.bitcast_convert_type(x, jnp.int32) >> 1)
        r = lax.bitcast_convert_type(i, jnp.float32)
        r = r * (1.5 - 0.5 * x * r * r)
        return x * r

    def compute(j, carry):
        for t in range(8):
            v = g_v[pl.ds(j * 128 + t * 16, 16)]
            da = sqrt16(lax.bitcast_convert_type(v << 16, jnp.float32))
            dl = sqrt16(lax.bitcast_convert_type(v & jnp.uint32(HIMASK),
                                                 jnp.float32))
            ra = jnp.maximum(MARGIN - da, 0.0)
            rl = jnp.maximum(MARGIN - dl, 0.0)
            if t == 0:
                posv = lax.iota(jnp.int32, 16) + ((j & 7) << 7)
                ra = jnp.where(posv == 0, da, ra)
                rl = jnp.where(posv == 0, dl, rl)
            ol_v[j, pl.ds(t * 16, 16)] = ra
            oab_v[j, pl.ds(t * 16, 16)] = rl
        return carry

    for h in range(4):
        lax.fori_loop(h * half, (h + 1) * half, make_fire(sems[h]), 0)
    for h in range(4):
        pltpu.make_async_copy(w_ref.at[pl.ds(0, half * 128)],
                              g_v.at[pl.ds(h * half * 128, half * 128)],
                              sems[h]).wait()
        lax.fori_loop(h * half, (h + 1) * half, compute, 0)
    pltpu.sync_copy(ol_v, ol_ref.at[wid])
    pltpu.sync_copy(oab_v, oab_ref.at[wid])


def _score(w_flat, idx4):
    mesh = plsc.VectorSubcoreMesh(core_axis_name="c", subcore_axis_name="s")
    f = pl.kernel(
        _score_body,
        out_type=(jax.ShapeDtypeStruct((NW, ROWS, 128), jnp.float32),
                  jax.ShapeDtypeStruct((NW, ROWS, 128), jnp.float32)),
        mesh=mesh,
        scratch_types=[
            pltpu.VMEM((ROWS, 128), jnp.int32),
            pltpu.VMEM((ROWS, 128), jnp.int32),
            pltpu.VMEM((ROWS * 128,), jnp.uint32),
            pltpu.VMEM((ROWS, 128), jnp.float32),
            pltpu.VMEM((ROWS, 128), jnp.float32),
            pltpu.SemaphoreType.DMA,
            pltpu.SemaphoreType.DMA,
            pltpu.SemaphoreType.DMA,
            pltpu.SemaphoreType.DMA,
        ],
    )
    return f(w_flat, idx4)


def kernel(l, ab, y, idx, memory_l, memory_ab):
    idx4 = idx.astype(jnp.int32).reshape(NW, ROWS, 128)
    w_flat = _pack(l, ab, memory_l, memory_ab)
    ol, oab = _score(w_flat, idx4)
    return (ol.reshape(B, KP1, 1), oab.reshape(B, KP1, 1))
